# baseline (speedup 1.0000x reference)
import functools

import jax
import jax.numpy as jnp
from jax import lax
from jax.experimental import pallas as pl
from jax.experimental.pallas import tpu as pltpu
from jax.experimental.pallas import tpu_sc as plsc

VOCAB = 100000
D_MODEL = 1024
SCALE = 32.0

NUM_CORES = 2
NUM_SUBCORES = 16
NUM_WORKERS = NUM_CORES * NUM_SUBCORES
LANES = 16

N_TOKENS = 4 * 4096
TOK_PER_WORKER = N_TOKENS // NUM_WORKERS
CHUNK = 32
N_CHUNKS = TOK_PER_WORKER // CHUNK
VECS_PER_ROW = D_MODEL // LANES


def _body(ids_hbm, table_hbm, out_hbm, idx_v, rows_v, sem):
  c = lax.axis_index("c")
  s = lax.axis_index("s")
  wid = s * NUM_CORES + c
  base = wid * TOK_PER_WORKER

  pltpu.sync_copy(ids_hbm.at[pl.ds(base, TOK_PER_WORKER)], idx_v)

  def chunk_step(g, carry):
    off = g * CHUNK
    pltpu.async_copy(
        table_hbm.at[idx_v.at[pl.ds(off, CHUNK)]], rows_v, sem
    ).wait()

    def row_step(r, carry2):
      def vec_step(j, carry3):
        sl = pl.ds(j * LANES, LANES)
        rows_v[r, sl] = rows_v[r, sl] * SCALE
        return carry3

      return lax.fori_loop(0, VECS_PER_ROW, vec_step, carry2)

    lax.fori_loop(0, CHUNK, row_step, 0)

    pltpu.sync_copy(rows_v, out_hbm.at[pl.ds(base + off, CHUNK)])
    return carry

  lax.fori_loop(0, N_CHUNKS, chunk_step, 0)


@jax.jit
def _embed(ids, table):
  mesh = plsc.VectorSubcoreMesh(core_axis_name="c", subcore_axis_name="s")
  return pl.kernel(
      _body,
      out_type=jax.ShapeDtypeStruct((N_TOKENS, D_MODEL), jnp.float32),
      mesh=mesh,
      scratch_types=[
          pltpu.VMEM((TOK_PER_WORKER,), jnp.int32),
          pltpu.VMEM((CHUNK, D_MODEL), jnp.float32),
          pltpu.SemaphoreType.DMA,
      ],
  )(ids, table)


def kernel(token_ids, embedding):
  ids = token_ids.reshape(-1).astype(jnp.int32)
  out = _embed(ids, embedding)
  return out.reshape(token_ids.shape + (D_MODEL,))


# speedup vs baseline: 3.3186x; 3.3186x over previous
import jax
import jax.numpy as jnp
from jax import lax
from jax.experimental import pallas as pl
from jax.experimental.pallas import tpu as pltpu
from jax.experimental.pallas import tpu_sc as plsc

VOCAB = 100000
D_MODEL = 1024
SCALE = 32.0

NUM_CORES = 2
NUM_SUBCORES = 16
NUM_WORKERS = NUM_CORES * NUM_SUBCORES
LANES = 16

N_TOKENS = 4 * 4096
TOK_PER_WORKER = N_TOKENS // NUM_WORKERS
CHUNK = 16
N_CHUNKS = TOK_PER_WORKER // CHUNK
NBUF = 4
VECS_PER_ROW = D_MODEL // LANES


def _body(ids_hbm, table_hbm, out_hbm, idx_v, b0, b1, b2, b3, *sems):
  bufs = [b0, b1, b2, b3]
  gsem = sems[:NBUF]
  ssem = sems[NBUF:]

  c = lax.axis_index("c")
  s = lax.axis_index("s")
  wid = s * NUM_CORES + c
  base = wid * TOK_PER_WORKER

  pltpu.sync_copy(ids_hbm.at[pl.ds(base, TOK_PER_WORKER)], idx_v)

  def gather(h, b):
    pltpu.make_async_copy(
        table_hbm.at[idx_v.at[pl.ds(h * CHUNK, CHUNK)]], bufs[b], gsem[b]
    ).start()

  def wait_gather(b):
    pltpu.make_async_copy(
        table_hbm.at[idx_v.at[pl.ds(0, CHUNK)]], bufs[b], gsem[b]
    ).wait()

  def scatter(h, b):
    pltpu.make_async_copy(
        bufs[b], out_hbm.at[pl.ds(base + h * CHUNK, CHUNK)], ssem[b]
    ).start()

  def wait_scatter(b):
    pltpu.make_async_copy(
        bufs[b], out_hbm.at[pl.ds(base, CHUNK)], ssem[b]
    ).wait()

  def scale(b):
    ref = bufs[b]

    def row_step(r, carry):
      for j in range(VECS_PER_ROW):
        sl = pl.ds(j * LANES, LANES)
        ref[r, sl] = ref[r, sl] * SCALE
      return carry

    lax.fori_loop(0, CHUNK, row_step, 0)

  gather(0, 0)
  gather(1, 1)

  wait_gather(0)
  scale(0)
  scatter(0, 0)
  gather(2, 2)
  wait_gather(1)
  scale(1)
  scatter(1, 1)
  gather(3, 3)

  def outer(o, carry):
    h0 = 2 + o * NBUF
    for k in range(NBUF):
      h = h0 + k
      b = (2 + k) % NBUF
      wait_gather(b)
      scale(b)
      scatter(h, b)
      b2 = k % NBUF
      wait_scatter(b2)
      gather(h + 2, b2)
    return carry

  lax.fori_loop(0, (N_CHUNKS - 4) // NBUF, outer, 0)

  wait_gather(2)
  scale(2)
  scatter(N_CHUNKS - 2, 2)
  wait_scatter(0)
  wait_gather(3)
  scale(3)
  scatter(N_CHUNKS - 1, 3)
  wait_scatter(1)
  wait_scatter(2)
  wait_scatter(3)


@jax.jit
def _embed(ids, table):
  mesh = plsc.VectorSubcoreMesh(core_axis_name="c", subcore_axis_name="s")
  return pl.kernel(
      _body,
      out_type=jax.ShapeDtypeStruct((N_TOKENS, D_MODEL), jnp.float32),
      mesh=mesh,
      scratch_types=[pltpu.VMEM((TOK_PER_WORKER,), jnp.int32)]
      + [pltpu.VMEM((CHUNK, D_MODEL), jnp.float32) for _ in range(NBUF)]
      + [pltpu.SemaphoreType.DMA for _ in range(2 * NBUF)],
  )(ids, table)


def kernel(token_ids, embedding):
  ids = token_ids.reshape(-1).astype(jnp.int32)
  out = _embed(ids, embedding)
  return out.reshape(token_ids.shape + (D_MODEL,))
